# baseline (speedup 1.0000x reference)
import functools

import jax
import jax.numpy as jnp
from jax import lax
from jax.experimental import pallas as pl
from jax.experimental.pallas import tpu as pltpu
from jax.experimental.pallas import tpu_sc as plsc

T = 16384
L = 16
WORD_DIM = 64
CHAR_VOCAB = 100
CHAR_DIM = 30
HID = 50
CTX_DIM = 128
OUT_DIM = WORD_DIM + 2 * HID + CTX_DIM
G = 128
NG = 4 * G
KX = 3 * 128

_SC_CORES = 2
_SC_SUBCORES = 16
_NW = _SC_CORES * _SC_SUBCORES


def _sc_word_gather(word_table, word):
    b_per_w = T // _NW
    mesh = plsc.VectorSubcoreMesh(core_axis_name="c", subcore_axis_name="s")

    @functools.partial(
        pl.kernel,
        mesh=mesh,
        compiler_params=pltpu.CompilerParams(use_tc_tiling_on_sc=False),
        out_type=jax.ShapeDtypeStruct((T, WORD_DIM), jnp.float32),
        scratch_types=[
            pltpu.VMEM((b_per_w,), jnp.int32),
            pltpu.VMEM((b_per_w, WORD_DIM), jnp.float32),
            pltpu.SemaphoreType.DMA,
        ],
    )
    def gather_kernel(table_hbm, idx_hbm, out_hbm, idx_v, rows_v, sem):
        wid = lax.axis_index("s") * _SC_CORES + lax.axis_index("c")
        base = wid * b_per_w
        pltpu.sync_copy(idx_hbm.at[pl.ds(base, b_per_w)], idx_v)
        pltpu.async_copy(table_hbm.at[idx_v], rows_v, sem).wait()
        pltpu.sync_copy(rows_v, out_hbm.at[pl.ds(base, b_per_w)])

    return gather_kernel(word_table, word)


def _lstm_block_body(char_ref, ctx_ref, ctab_ref,
                     wi_ref, wh_ref, b_ref, out_ref):
    B = char_ref.shape[0]
    f32 = jnp.float32
    bf16 = jnp.bfloat16
    dot = functools.partial(jnp.dot, precision=jax.lax.Precision.DEFAULT,
                            preferred_element_type=f32)
    dot_hi = functools.partial(jnp.dot, precision=jax.lax.Precision.HIGHEST,
                               preferred_element_type=f32)

    col = lax.broadcasted_iota(jnp.int32, (1, NG), 1)
    is_g = jnp.logical_and(col >= 2 * G, col < 3 * G)
    scale = jnp.where(is_g, 1.0, 0.5).astype(f32)

    ctab = ctab_ref[...]
    E = dot_hi(ctab, wi_ref[...]) + b_ref[...]
    M = jnp.concatenate([E[:128], wh_ref[...], E[128:]], axis=0)
    M = (M * scale).astype(bf16)

    NC = 4
    BH = B // NC
    chars_all = char_ref[...]
    vocab_iota = lax.broadcasted_iota(jnp.int32, (BH, 128), 1)

    def onehot(chars, t):
        c = chars[:, t:t + 1]
        return (vocab_iota == c).astype(bf16)

    chains = []
    for k in range(NC):
        chains.append({
            "chars": chars_all[k * BH:(k + 1) * BH],
            "h": jnp.zeros((BH, G), f32),
            "c": jnp.zeros((BH, G), f32),
        })
    for t in range(L):
        for s in chains:
            s["xt"] = jnp.concatenate(
                [onehot(s["chars"], t), s["h"].astype(bf16),
                 onehot(s["chars"], L - 1 - t)], axis=-1)
        for s in chains:
            s["tg"] = jnp.tanh(dot(s["xt"], M))
        for s in chains:
            tg = s["tg"]
            i_s = 0.5 + 0.5 * tg[:, 0:G]
            f_s = 0.5 + 0.5 * tg[:, G:2 * G]
            g_t = tg[:, 2 * G:3 * G]
            o_s = 0.5 + 0.5 * tg[:, 3 * G:4 * G]
            s["c"] = f_s * s["c"] + i_s * g_t
            s["h"] = o_s * jnp.tanh(s["c"])

    h = jnp.concatenate([s["h"] for s in chains], axis=0)
    out_ref[...] = jnp.concatenate(
        [jnp.zeros((B, WORD_DIM), f32), h[:, :HID], h[:, 64:64 + HID],
         ctx_ref[...]], axis=-1)


def _place_gate_cols(w, off):
    lead = w.shape[:-1]
    w4 = w.reshape(lead + (4, HID))
    pad = [(0, 0)] * len(lead) + [(0, 0), (off, G - HID - off)]
    return jnp.pad(w4, pad).reshape(lead + (NG,))


def _tc_forward(char, ctx, char_table,
                Wi_f, Wh_f, b_f, Wi_b, Wh_b, b_b, interpret=False):
    BT = 1024
    grid = (T // BT,)
    blk = lambda r, c: pl.BlockSpec((r, c), lambda i: (i, 0))
    full = lambda r, c: pl.BlockSpec((r, c), lambda i: (0, 0))

    wi_both = jnp.pad(jnp.concatenate(
        [_place_gate_cols(Wi_f, 0), _place_gate_cols(Wi_b, 64)], axis=0),
        ((0, 4), (0, 0)))
    wh_both = jnp.concatenate([
        jnp.pad(_place_gate_cols(Wh_f, 0), ((0, 64 - HID), (0, 0))),
        jnp.pad(_place_gate_cols(Wh_b, 64), ((0, 64 - HID), (0, 0))),
    ], axis=0)
    bf_row = _place_gate_cols(b_f, 0).reshape(1, NG)
    bb_row = _place_gate_cols(b_b, 64).reshape(1, NG)
    b_exp = jnp.concatenate([
        jnp.broadcast_to(bf_row, (CHAR_VOCAB, NG)),
        jnp.zeros((128 - CHAR_VOCAB, NG), jnp.float32),
        jnp.broadcast_to(bb_row, (CHAR_VOCAB, NG)),
        jnp.zeros((128 - CHAR_VOCAB, NG), jnp.float32),
    ], axis=0)
    zc = jnp.zeros_like(char_table)
    zrow = jnp.zeros((128 - CHAR_VOCAB, 2 * CHAR_DIM), jnp.float32)
    ctab2 = jnp.pad(jnp.concatenate([
        jnp.concatenate([char_table, zc], axis=1), zrow,
        jnp.concatenate([zc, char_table], axis=1), zrow,
    ], axis=0), ((0, 0), (0, 4)))

    return pl.pallas_call(
        _lstm_block_body,
        grid=grid,
        in_specs=[
            blk(BT, L),
            blk(BT, CTX_DIM),
            full(256, 64),
            full(64, NG),
            full(G, NG),
            full(256, NG),
        ],
        out_specs=blk(BT, OUT_DIM),
        out_shape=jax.ShapeDtypeStruct((T, OUT_DIM), jnp.float32),
        compiler_params=pltpu.CompilerParams(
            dimension_semantics=("parallel",)),
        interpret=interpret,
    )(char, ctx, ctab2, wi_both, wh_both, b_exp)


def _merge_word_body(big_ref, wemb_ref, out_ref):
    out_ref[...] = jnp.concatenate(
        [wemb_ref[...], big_ref[:, WORD_DIM:]], axis=-1)


def _merge_word(big, word_emb, interpret=False):
    BT = 2048
    return pl.pallas_call(
        _merge_word_body,
        grid=(T // BT,),
        in_specs=[
            pl.BlockSpec((BT, 128), lambda i: (i, 0)),
            pl.BlockSpec((BT, WORD_DIM), lambda i: (i, 0)),
        ],
        out_specs=pl.BlockSpec((BT, 128), lambda i: (i, 0)),
        out_shape=jax.ShapeDtypeStruct((T, OUT_DIM), jnp.float32),
        input_output_aliases={0: 0},
        compiler_params=pltpu.CompilerParams(
            dimension_semantics=("parallel",)),
        interpret=interpret,
    )(big, word_emb)


def kernel(word, char, ctx, word_table, char_table,
           Wi_f, Wh_f, b_f, Wi_b, Wh_b, b_b):
    word_emb = _sc_word_gather(word_table, word.astype(jnp.int32))
    big = _tc_forward(char, ctx, char_table,
                      Wi_f, Wh_f, b_f, Wi_b, Wh_b, b_b)
    return _merge_word(big, word_emb)


# speedup vs baseline: 1.0435x; 1.0435x over previous
import functools

import jax
import jax.numpy as jnp
from jax import lax
from jax.experimental import pallas as pl
from jax.experimental.pallas import tpu as pltpu
from jax.experimental.pallas import tpu_sc as plsc

T = 16384
L = 16
WORD_DIM = 64
CHAR_VOCAB = 100
CHAR_DIM = 30
HID = 50
CTX_DIM = 128
OUT_DIM = WORD_DIM + 2 * HID + CTX_DIM
G = 128
NG = 4 * G
KX = 3 * 128
WORD_VOCAB_HALF = 50000

_SC_CORES = 2
_SC_SUBCORES = 16
_NW = _SC_CORES * _SC_SUBCORES


def _sc_word_gather(word_table, word):
    b_per_w = T // _NW
    mesh = plsc.VectorSubcoreMesh(core_axis_name="c", subcore_axis_name="s")

    @functools.partial(
        pl.kernel,
        mesh=mesh,
        out_type=jax.ShapeDtypeStruct((T, 2 * WORD_DIM), jnp.float32),
        scratch_types=[
            pltpu.VMEM((b_per_w,), jnp.int32),
            pltpu.VMEM((b_per_w, 2 * WORD_DIM), jnp.float32),
            pltpu.SemaphoreType.DMA,
        ],
    )
    def gather_kernel(table_hbm, idx_hbm, out_hbm, idx_v, rows_v, sem):
        wid = lax.axis_index("s") * _SC_CORES + lax.axis_index("c")
        base = wid * b_per_w
        pltpu.sync_copy(idx_hbm.at[pl.ds(base, b_per_w)], idx_v)
        pltpu.async_copy(table_hbm.at[idx_v], rows_v, sem).wait()
        pltpu.sync_copy(rows_v, out_hbm.at[pl.ds(base, b_per_w)])

    return gather_kernel(word_table, word)


def _lstm_block_body(char_ref, wemb_ref, wpar_ref, ctx_ref, ctab_ref,
                     wi_ref, wh_ref, b_ref, out_ref):
    B = char_ref.shape[0]
    f32 = jnp.float32
    bf16 = jnp.bfloat16
    dot = functools.partial(jnp.dot, precision=jax.lax.Precision.DEFAULT,
                            preferred_element_type=f32)
    dot_hi = functools.partial(jnp.dot, precision=jax.lax.Precision.HIGHEST,
                               preferred_element_type=f32)

    col = lax.broadcasted_iota(jnp.int32, (1, NG), 1)
    is_g = jnp.logical_and(col >= 2 * G, col < 3 * G)
    scale = jnp.where(is_g, 1.0, 0.5).astype(f32)

    ctab = ctab_ref[...]
    E = dot_hi(ctab, wi_ref[...]) + b_ref[...]
    M = jnp.concatenate([E[:128], wh_ref[...], E[128:]], axis=0)
    M = (M * scale).astype(bf16)

    NC = 4
    BH = B // NC
    chars_all = char_ref[...]
    vocab_iota = lax.broadcasted_iota(jnp.int32, (BH, 128), 1)

    def onehot(chars, t):
        c = chars[:, t:t + 1]
        return (vocab_iota == c).astype(bf16)

    chains = []
    for k in range(NC):
        chains.append({
            "chars": chars_all[k * BH:(k + 1) * BH],
            "h": jnp.zeros((BH, G), f32),
            "c": jnp.zeros((BH, G), f32),
        })
    for t in range(L):
        for s in chains:
            s["xt"] = jnp.concatenate(
                [onehot(s["chars"], t), s["h"].astype(bf16),
                 onehot(s["chars"], L - 1 - t)], axis=-1)
        for s in chains:
            s["tg"] = jnp.tanh(dot(s["xt"], M))
        for s in chains:
            tg = s["tg"]
            i_s = 0.5 + 0.5 * tg[:, 0:G]
            f_s = 0.5 + 0.5 * tg[:, G:2 * G]
            g_t = tg[:, 2 * G:3 * G]
            o_s = 0.5 + 0.5 * tg[:, 3 * G:4 * G]
            s["c"] = f_s * s["c"] + i_s * g_t
            s["h"] = o_s * jnp.tanh(s["c"])

    h = jnp.concatenate([s["h"] for s in chains], axis=0)
    rows = wemb_ref[...]
    even = wpar_ref[...] == 0
    wemb = jnp.where(even, rows[:, :WORD_DIM], rows[:, WORD_DIM:])
    out_ref[...] = jnp.concatenate(
        [wemb, h[:, :HID], h[:, 64:64 + HID], ctx_ref[...]],
        axis=-1)


def _place_gate_cols(w, off):
    lead = w.shape[:-1]
    w4 = w.reshape(lead + (4, HID))
    pad = [(0, 0)] * len(lead) + [(0, 0), (off, G - HID - off)]
    return jnp.pad(w4, pad).reshape(lead + (NG,))


def _tc_forward(word_emb, word_par, char, ctx, char_table,
                Wi_f, Wh_f, b_f, Wi_b, Wh_b, b_b, interpret=False):
    BT = 1024
    grid = (T // BT,)
    blk = lambda r, c: pl.BlockSpec((r, c), lambda i: (i, 0))
    full = lambda r, c: pl.BlockSpec((r, c), lambda i: (0, 0))

    wi_both = jnp.pad(jnp.concatenate(
        [_place_gate_cols(Wi_f, 0), _place_gate_cols(Wi_b, 64)], axis=0),
        ((0, 4), (0, 0)))
    wh_both = jnp.concatenate([
        jnp.pad(_place_gate_cols(Wh_f, 0), ((0, 64 - HID), (0, 0))),
        jnp.pad(_place_gate_cols(Wh_b, 64), ((0, 64 - HID), (0, 0))),
    ], axis=0)
    bf_row = _place_gate_cols(b_f, 0).reshape(1, NG)
    bb_row = _place_gate_cols(b_b, 64).reshape(1, NG)
    b_exp = jnp.concatenate([
        jnp.broadcast_to(bf_row, (CHAR_VOCAB, NG)),
        jnp.zeros((128 - CHAR_VOCAB, NG), jnp.float32),
        jnp.broadcast_to(bb_row, (CHAR_VOCAB, NG)),
        jnp.zeros((128 - CHAR_VOCAB, NG), jnp.float32),
    ], axis=0)
    zc = jnp.zeros_like(char_table)
    zrow = jnp.zeros((128 - CHAR_VOCAB, 2 * CHAR_DIM), jnp.float32)
    ctab2 = jnp.pad(jnp.concatenate([
        jnp.concatenate([char_table, zc], axis=1), zrow,
        jnp.concatenate([zc, char_table], axis=1), zrow,
    ], axis=0), ((0, 0), (0, 4)))

    return pl.pallas_call(
        _lstm_block_body,
        grid=grid,
        in_specs=[
            blk(BT, L),
            blk(BT, 2 * WORD_DIM),
            blk(BT, 1),
            blk(BT, CTX_DIM),
            full(256, 64),
            full(64, NG),
            full(G, NG),
            full(256, NG),
        ],
        out_specs=blk(BT, OUT_DIM),
        out_shape=jax.ShapeDtypeStruct((T, OUT_DIM), jnp.float32),
        compiler_params=pltpu.CompilerParams(
            dimension_semantics=("parallel",)),
        interpret=interpret,
    )(char, word_emb, word_par, ctx, ctab2, wi_both, wh_both, b_exp)


def kernel(word, char, ctx, word_table, char_table,
           Wi_f, Wh_f, b_f, Wi_b, Wh_b, b_b):
    word = word.astype(jnp.int32)
    table2 = word_table.reshape(WORD_VOCAB_HALF, 2 * WORD_DIM)
    word_emb = _sc_word_gather(table2, word >> 1)
    return _tc_forward(word_emb, (word & 1).reshape(T, 1), char, ctx,
                       char_table,
                       Wi_f, Wh_f, b_f, Wi_b, Wh_b, b_b)
